# Initial kernel scaffold; baseline (speedup 1.0000x reference)
import jax
import jax.numpy as jnp
from jax.experimental import pallas as pl


def kernel(node_in, node_mask, edge_index_0, edge_index_1, edge_index_2, m_id_0, m_id_1, params, norm_stats):
    raise NotImplementedError("write your pallas kernel here")


# speedup vs baseline: 1.1195x; 1.1195x over previous
import functools

import jax
import jax.numpy as jnp
from jax import lax
from jax.experimental import pallas as pl
from jax.experimental.pallas import tpu as pltpu
from jax.experimental.pallas import tpu_sc as plsc

_LAT = 128
_NC, _NS = 2, 16
_NW = _NC * _NS
_CH = 128
_BN = 512
_BE = 512
_LN_EPS = 1e-5
_INTERP = False


def _rup(x, m):
    return (x + m - 1) // m * m


def _sc_gather(tables, idxs, pairs, B):
    nt, ni, np_ = len(tables), len(idxs), len(pairs)
    W = B // _NW
    nch = W // _CH
    assert W % _CH == 0, (B, W)
    mesh = plsc.VectorSubcoreMesh(core_axis_name="c", subcore_axis_name="s", num_cores=_NC, num_subcores=_NS)
    out_type = tuple(
        jax.ShapeDtypeStruct((B, tables[j].shape[1]), jnp.float32)
        for (_, j) in pairs)
    scratch = (
        [pltpu.VMEM((1, _CH), jnp.int32) for _ in range(ni)]
        + [pltpu.VMEM((_CH, tables[j].shape[1]), jnp.float32)
           for (_, j) in pairs]
        + [pltpu.SemaphoreType.DMA])

    def body(*refs):
        tabs = refs[:nt]
        idxr = refs[nt:nt + ni]
        outs = refs[nt + ni:nt + ni + np_]
        ibufs = refs[nt + ni + np_:nt + ni + np_ + ni]
        gbufs = refs[nt + ni + np_ + ni:nt + ni + np_ + ni + np_]
        sem = refs[-1]
        wid = lax.axis_index("s") * _NC + lax.axis_index("c")

        def chunk(k, carry):
            base = wid * W + k * _CH
            row = base // _CH
            for i in range(ni):
                pltpu.sync_copy(idxr[i].at[pl.ds(row, 1)], ibufs[i])
            cps = []
            for p, (i, j) in enumerate(pairs):
                cps.append(pltpu.async_copy(
                    tabs[j].at[ibufs[i].at[0]], gbufs[p], sem))
            for cp in cps:
                cp.wait()
            for p in range(np_):
                pltpu.sync_copy(gbufs[p], outs[p].at[pl.ds(base, _CH)])
            return carry

        lax.fori_loop(0, nch, chunk, 0)

    fn = pl.kernel(body, out_type=out_type, mesh=mesh, scratch_types=scratch,
                   compiler_params=pltpu.CompilerParams(
                       use_tc_tiling_on_sc=False),
                   interpret=_INTERP)
    res = fn(*tables, *idxs)
    return list(res) if isinstance(res, (tuple, list)) else [res]


def _sc_scatter_add(e, ridx, zeros128, Np):
    B = e.shape[0]
    W = B // _NW
    nch = W // _CH
    assert W % _CH == 0 and Np % (_NS * _CH) == 0
    rpt = Np // _NS
    mesh = plsc.VectorSubcoreMesh(core_axis_name="c", subcore_axis_name="s", num_cores=_NC, num_subcores=_NS)
    out_type = jax.ShapeDtypeStruct((2, Np, _LAT), jnp.float32)
    scratch = (pltpu.VMEM((1, _CH), jnp.int32),
               pltpu.VMEM((_CH, _LAT), jnp.float32),
               pltpu.VMEM_SHARED((Np, _LAT), jnp.float32))

    def body(e_ref, idx_ref, z_ref, out_ref, ibuf, ebuf, acc):
        c = lax.axis_index("c")
        s = lax.axis_index("s")
        for k in range(rpt // _CH):
            pltpu.sync_copy(z_ref, acc.at[pl.ds(s * rpt + k * _CH, _CH)])
        plsc.subcore_barrier()
        wid = s * _NC + c

        def chunk(k, carry):
            base = wid * W + k * _CH
            row = base // _CH
            pltpu.sync_copy(idx_ref.at[pl.ds(row, 1)], ibuf)
            pltpu.sync_copy(e_ref.at[pl.ds(base, _CH)], ebuf)
            pltpu.sync_copy(ebuf, acc.at[ibuf.at[0]], add=True)
            return carry

        lax.fori_loop(0, nch, chunk, 0)
        plsc.subcore_barrier()
        for k in range(rpt // _CH):
            off = s * rpt + k * _CH
            pltpu.sync_copy(acc.at[pl.ds(off, _CH)],
                            out_ref.at[c].at[pl.ds(off, _CH)])

    fn = pl.kernel(body, out_type=out_type, mesh=mesh, scratch_types=scratch,
                   interpret=_INTERP)
    return fn(e, ridx, zeros128)


def _row_spec(b, d):
    return pl.BlockSpec((b, d), lambda i: (i, 0))


def _full_spec(shape):
    return pl.BlockSpec(shape, lambda i: tuple(0 for _ in shape))


def _dot(a, b):
    return jnp.dot(a, b, preferred_element_type=jnp.float32)


def _ln(x, g, bt):
    m = jnp.mean(x, axis=-1, keepdims=True)
    v = jnp.mean((x - m) * (x - m), axis=-1, keepdims=True)
    return (x - m) * lax.rsqrt(v + _LN_EPS) * g + bt


def _tc_encode(x4p, im, isd, W1, b1, W2, b2, W3, b3, g, bt, N):
    Np = x4p.shape[0]

    def body(x_ref, im_r, is_r, w1, b1r, w2, b2r, w3, b3r, gr, btr, o_ref):
        x = (x_ref[...] - im_r[...]) / is_r[...]
        x = jnp.maximum(_dot(x, w1[...]) + b1r[...], 0.0)
        x = jnp.maximum(_dot(x, w2[...]) + b2r[...], 0.0)
        x = _ln(_dot(x, w3[...]) + b3r[...], gr[...], btr[...])
        rid = pl.program_id(0) * _BN + lax.broadcasted_iota(
            jnp.int32, (_BN, _LAT), 0)
        o_ref[...] = jnp.where(rid < N, x, 0.0)

    return pl.pallas_call(
        body, grid=(Np // _BN,),
        in_specs=[_row_spec(_BN, _LAT), _full_spec((1, _LAT)),
                  _full_spec((1, _LAT)), _full_spec((_LAT, _LAT)),
                  _full_spec((1, _LAT)), _full_spec((_LAT, _LAT)),
                  _full_spec((1, _LAT)), _full_spec((_LAT, _LAT)),
                  _full_spec((1, _LAT)), _full_spec((1, _LAT)),
                  _full_spec((1, _LAT))],
        out_specs=_row_spec(_BN, _LAT),
        out_shape=jax.ShapeDtypeStruct((Np, _LAT), jnp.float32),
        interpret=_INTERP,
    )(x4p, im, isd, W1, b1, W2, b2, W3, b3, g, bt)


def _tc_edge(hs, hr, ps, pr, W1a, W1b, Wd, b1, W2, b2, W3, b3, g, bt):
    B = hs.shape[0]

    def body(hs_ref, hr_ref, ps_ref, pr_ref, w1a, w1b, wd, b1r, w2, b2r,
             w3, b3r, gr, btr, o_ref):
        d16 = ps_ref[...] - pr_ref[...]
        nrm = jnp.sqrt(jnp.sum(d16 * d16, axis=-1, keepdims=True) + 1e-12)
        col = lax.broadcasted_iota(jnp.int32, (_BE, 16), 1)
        dn = jnp.where(col == 2, nrm, d16)
        x = (_dot(hs_ref[...], w1a[...]) + _dot(hr_ref[...], w1b[...])
             + _dot(dn, wd[...]) + b1r[...])
        x = jnp.maximum(x, 0.0)
        x = jnp.maximum(_dot(x, w2[...]) + b2r[...], 0.0)
        x = _dot(x, w3[...]) + b3r[...]
        o_ref[...] = _ln(x, gr[...], btr[...])

    return pl.pallas_call(
        body, grid=(B // _BE,),
        in_specs=[_row_spec(_BE, _LAT), _row_spec(_BE, _LAT),
                  _row_spec(_BE, 16), _row_spec(_BE, 16),
                  _full_spec((_LAT, _LAT)), _full_spec((_LAT, _LAT)),
                  _full_spec((16, _LAT)), _full_spec((1, _LAT)),
                  _full_spec((_LAT, _LAT)), _full_spec((1, _LAT)),
                  _full_spec((_LAT, _LAT)), _full_spec((1, _LAT)),
                  _full_spec((1, _LAT)), _full_spec((1, _LAT))],
        out_specs=_row_spec(_BE, _LAT),
        out_shape=jax.ShapeDtypeStruct((B, _LAT), jnp.float32),
        interpret=_INTERP,
    )(hs, hr, ps, pr, W1a, W1b, Wd, b1, W2, b2, W3, b3, g, bt)


def _tc_node(h, agg, skip, W1a, W1b, b1, W2, b2, W3, b3, g, bt, N):
    Np = h.shape[0]
    has_skip = skip is not None

    def body(*refs):
        if has_skip:
            (h_ref, a0_ref, a1_ref, sk_ref, w1a, w1b, b1r, w2, b2r, w3, b3r,
             gr, btr, o_ref) = refs
        else:
            (h_ref, a0_ref, a1_ref, w1a, w1b, b1r, w2, b2r, w3, b3r,
             gr, btr, o_ref) = refs
        hv = h_ref[...]
        a = a0_ref[...] + a1_ref[...]
        x = jnp.maximum(_dot(hv, w1a[...]) + _dot(a, w1b[...]) + b1r[...], 0.0)
        x = jnp.maximum(_dot(x, w2[...]) + b2r[...], 0.0)
        x = _ln(_dot(x, w3[...]) + b3r[...], gr[...], btr[...])
        out = hv + x
        if has_skip:
            out = out + sk_ref[...]
        rid = pl.program_id(0) * _BN + lax.broadcasted_iota(
            jnp.int32, (_BN, _LAT), 0)
        o_ref[...] = jnp.where(rid < N, out, 0.0)

    ins = [h, agg[0], agg[1]] + ([skip] if has_skip else [])
    in_specs = [_row_spec(_BN, _LAT)] * len(ins) + [
        _full_spec((_LAT, _LAT)), _full_spec((_LAT, _LAT)),
        _full_spec((1, _LAT)), _full_spec((_LAT, _LAT)),
        _full_spec((1, _LAT)), _full_spec((_LAT, _LAT)),
        _full_spec((1, _LAT)), _full_spec((1, _LAT)), _full_spec((1, _LAT))]
    return pl.pallas_call(
        body, grid=(Np // _BN,),
        in_specs=in_specs,
        out_specs=_row_spec(_BN, _LAT),
        out_shape=jax.ShapeDtypeStruct((Np, _LAT), jnp.float32),
        interpret=_INTERP,
    )(*ins, W1a, W1b, b1, W2, b2, W3, b3, g, bt)


def _tc_decode(h, x3p, maskp, W1, b1, W2, b2, W3, b3, os_, om):
    Np = h.shape[0]

    def body(h_ref, x3_ref, m_ref, w1, b1r, w2, b2r, w3, b3r, osr, omr, o_ref):
        x = jnp.maximum(_dot(h_ref[...], w1[...]) + b1r[...], 0.0)
        x = jnp.maximum(_dot(x, w2[...]) + b2r[...], 0.0)
        x = _dot(x, w3[...]) + b3r[...]
        delta = (x * osr[...] + omr[...]) * m_ref[...]
        o_ref[...] = x3_ref[...] + delta

    return pl.pallas_call(
        body, grid=(Np // _BN,),
        in_specs=[_row_spec(_BN, _LAT), _row_spec(_BN, _LAT),
                  _row_spec(_BN, _LAT), _full_spec((_LAT, _LAT)),
                  _full_spec((1, _LAT)), _full_spec((_LAT, _LAT)),
                  _full_spec((1, _LAT)), _full_spec((_LAT, _LAT)),
                  _full_spec((1, _LAT)), _full_spec((1, _LAT)),
                  _full_spec((1, _LAT))],
        out_specs=_row_spec(_BN, _LAT),
        out_shape=jax.ShapeDtypeStruct((Np, _LAT), jnp.float32),
        interpret=_INTERP,
    )(h, x3p, maskp, W1, b1, W2, b2, W3, b3, os_, om)


def _r1(b):
    return b.reshape(1, -1)


def _prep_edge(p):
    W1 = p["Ws"][0]
    Wd = jnp.zeros((16, _LAT), jnp.float32).at[:3].set(W1[2 * _LAT:2 * _LAT + 3])
    return (W1[:_LAT], W1[_LAT:2 * _LAT], Wd, _r1(p["bs"][0]),
            p["Ws"][1], _r1(p["bs"][1]), p["Ws"][2], _r1(p["bs"][2]),
            _r1(p["g"]), _r1(p["bt"]))


def _prep_node(p):
    W1 = p["Ws"][0]
    return (W1[:_LAT], W1[_LAT:], _r1(p["bs"][0]),
            p["Ws"][1], _r1(p["bs"][1]), p["Ws"][2], _r1(p["bs"][2]),
            _r1(p["g"]), _r1(p["bt"]))


def _pad_rows(a, n, fill=0.0):
    return jnp.pad(a, ((0, n - a.shape[0]), (0, 0)), constant_values=fill)


def _pad_idx(a, n, fill):
    return jnp.pad(a, (0, n - a.shape[0]), constant_values=fill
                   ).astype(jnp.int32).reshape(-1, _CH)


def _inv_idx(m_id, Nf, Nc, Nfp):
    ar = jnp.arange(Nf, dtype=m_id.dtype)
    j = jnp.searchsorted(m_id, ar)
    jc = jnp.minimum(j, Nc - 1)
    inv = jnp.where(m_id[jc] == ar, jc, Nc).astype(jnp.int32)
    return _pad_idx(inv, Nfp, Nc)


def _gmp(p, h, P, s2, r2, z128, Np, N, skip=None):
    hs, hr, ps, pr = _sc_gather([h, P], [s2, r2],
                                [(0, 0), (1, 0), (0, 1), (1, 1)],
                                s2.shape[0] * _CH)
    e = _tc_edge(hs, hr, ps, pr, *_prep_edge(p["edge"]))
    agg = _sc_scatter_add(e, r2, z128, Np)
    return _tc_node(h, agg, skip, *_prep_node(p["node"]), N)


def kernel(node_in, node_mask, edge_index_0, edge_index_1, edge_index_2,
           m_id_0, m_id_1, params, norm_stats):
    N0 = node_in.shape[1]
    N1 = m_id_0.shape[0]
    N2 = m_id_1.shape[0]
    E0 = edge_index_0.shape[1]
    E1 = edge_index_1.shape[1]
    E2 = edge_index_2.shape[1]
    N0p, N1p, N2p = _rup(N0 + 1, 4096), _rup(N1 + 1, 4096), _rup(N2 + 1, 4096)
    E0p, E1p, E2p = _rup(E0, 4096), _rup(E1, 4096), _rup(E2, 4096)

    x = jnp.concatenate([node_in[0, :, :3], node_in[0, :, -1:]], axis=-1)
    pos = node_in[0, :, 3:5]
    x4p = _pad_rows(jnp.pad(x, ((0, 0), (0, _LAT - 4))), N0p)
    x3p = _pad_rows(jnp.pad(x[:, :3], ((0, 0), (0, _LAT - 3))), N0p)
    P0 = _pad_rows(jnp.pad(pos, ((0, 0), (0, 14))), N0p)
    maskp = _pad_rows(jnp.broadcast_to(node_mask[0], (N0, _LAT)), N0p)
    z128 = jnp.zeros((_CH, _LAT), jnp.float32)

    s0, r0 = (_pad_idx(edge_index_0[0], E0p, N0),
              _pad_idx(edge_index_0[1], E0p, N0))
    s1, r1 = (_pad_idx(edge_index_1[0], E1p, N1),
              _pad_idx(edge_index_1[1], E1p, N1))
    s2, r2 = (_pad_idx(edge_index_2[0], E2p, N2),
              _pad_idx(edge_index_2[1], E2p, N2))
    m0 = _pad_idx(m_id_0, N1p, N0)
    m1 = _pad_idx(m_id_1, N2p, N1)
    inv0 = _inv_idx(m_id_0, N0, N1, N0p)
    inv1 = _inv_idx(m_id_1, N1, N2, N1p)

    ns = norm_stats
    im = jnp.zeros((1, _LAT), jnp.float32).at[0, :4].set(ns["in_mean"])
    isd = jnp.ones((1, _LAT), jnp.float32).at[0, :4].set(ns["in_std"])
    om = jnp.zeros((1, _LAT), jnp.float32).at[0, :3].set(ns["out_mean"])
    osd = jnp.zeros((1, _LAT), jnp.float32).at[0, :3].set(ns["out_std"])

    enc = params["encode"]
    W1e = jnp.zeros((_LAT, _LAT), jnp.float32).at[:4].set(enc["Ws"][0])
    h0 = _tc_encode(x4p, im, isd, W1e, _r1(enc["bs"][0]), enc["Ws"][1],
                    _r1(enc["bs"][1]), enc["Ws"][2], _r1(enc["bs"][2]),
                    _r1(enc["g"]), _r1(enc["bt"]), N0)

    pr = params["process"]
    h0 = _gmp(pr["down"][0], h0, P0, s0, r0, z128, N0p, N0)
    down0 = h0
    h1, P1 = _sc_gather([h0, P0], [m0], [(0, 0), (0, 1)], N1p)
    h1 = _gmp(pr["down"][1], h1, P1, s1, r1, z128, N1p, N1)
    down1 = h1
    h2, P2 = _sc_gather([h1, P1], [m1], [(0, 0), (0, 1)], N2p)
    h2 = _gmp(pr["bottom"], h2, P2, s2, r2, z128, N2p, N2)

    hu1 = _sc_gather([h2], [inv1], [(0, 0)], N1p)[0]
    h1b = _gmp(pr["up"][0], hu1, P1, s1, r1, z128, N1p, N1, skip=down1)
    hu0 = _sc_gather([h1b], [inv0], [(0, 0)], N0p)[0]
    h0b = _gmp(pr["up"][1], hu0, P0, s0, r0, z128, N0p, N0, skip=down0)

    dec = params["decode"]
    W3d = jnp.zeros((_LAT, _LAT), jnp.float32).at[:, :3].set(dec["Ws"][2])
    b3d = jnp.zeros((1, _LAT), jnp.float32).at[0, :3].set(dec["bs"][2])
    out = _tc_decode(h0b, x3p, maskp, dec["Ws"][0], _r1(dec["bs"][0]),
                     dec["Ws"][1], _r1(dec["bs"][1]), W3d, b3d, osd, om)
    return out[:N0, :3][None]
